# baseline (speedup 1.0000x reference)
import jax
import jax.numpy as jnp
from jax import lax
from jax.experimental import pallas as pl
from jax.experimental.pallas import tpu as pltpu
from jax.experimental.pallas import tpu_sc as plsc

B = 16384
D = 50

_info = plsc.get_sparse_core_info()
_NC, _NS = _info.num_cores, _info.num_subcores
_NW = _NC * _NS
_BPW = B // _NW

_mesh = plsc.VectorSubcoreMesh(core_axis_name="c", subcore_axis_name="s")


def _sc_gather_body(u_idx, n_idx, c_idx, s_idx, e_idx,
                    ut, nt, ct, st, et,
                    ou, on, oc, osub, oe,
                    idx_v, rows_v, sem):
    wid = lax.axis_index("s") * _NC + lax.axis_index("c")
    base = wid * _BPW
    for idx_hbm, tab, out in ((u_idx, ut, ou), (n_idx, nt, on),
                              (c_idx, ct, oc), (s_idx, st, osub),
                              (e_idx, et, oe)):
        pltpu.sync_copy(idx_hbm.at[pl.ds(base, _BPW)], idx_v)

        def issue(g, _):
            v = idx_v[pl.ds(g * 16, 16)]
            for k in range(16):
                pltpu.async_copy(tab.at[pl.ds(v[k], 1)],
                                 rows_v.at[pl.ds(g * 16 + k, 1)], sem)
            return 0

        lax.fori_loop(0, _BPW // 16, issue, 0)
        pltpu.make_async_copy(out.at[pl.ds(base, _BPW)], rows_v, sem).wait()
        pltpu.sync_copy(rows_v, out.at[pl.ds(base, _BPW)])


_sc_gather = pl.kernel(
    _sc_gather_body,
    mesh=_mesh,
    out_type=[jax.ShapeDtypeStruct((B, D), jnp.float32)] * 5,
    scratch_types=[
        pltpu.VMEM((_BPW,), jnp.int32),
        pltpu.VMEM((_BPW, D), jnp.float32),
        pltpu.SemaphoreType.DMA,
    ],
)

_BM = 2048


def _tc_body(u_ref, n_ref, c_ref, s_ref, e_ref, w_ref, b_ref, o_ref):
    w = w_ref[...]
    z = jnp.dot(n_ref[...], w[0:D], preferred_element_type=jnp.float32)
    z = z + jnp.dot(c_ref[...], w[D:2 * D], preferred_element_type=jnp.float32)
    z = z + jnp.dot(s_ref[...], w[2 * D:3 * D], preferred_element_type=jnp.float32)
    z = z + jnp.dot(e_ref[...], w[3 * D:4 * D], preferred_element_type=jnp.float32)
    z = z + b_ref[...]
    sg = 1.0 / (1.0 + jnp.exp(-z))
    d = jnp.sum(u_ref[...] * sg, axis=1, keepdims=True)
    o_ref[...] = 1.0 / (1.0 + jnp.exp(-d))


_tc_fuse = pl.pallas_call(
    _tc_body,
    grid=(B // _BM,),
    in_specs=[pl.BlockSpec((_BM, D), lambda i: (i, 0))] * 5 + [
        pl.BlockSpec((4 * D, D), lambda i: (0, 0)),
        pl.BlockSpec((1, D), lambda i: (0, 0)),
    ],
    out_specs=pl.BlockSpec((_BM, 1), lambda i: (i, 0)),
    out_shape=jax.ShapeDtypeStruct((B, 1), jnp.float32),
)


def kernel(users, items, categories, subcategories, entities,
           user_table, news_table, cat_table, subcat_table, entity_table, W, b):
    u = users.astype(jnp.int32)
    n = items.astype(jnp.int32)
    c = categories.astype(jnp.int32)
    s = subcategories.astype(jnp.int32)
    e = entities[:, 0].astype(jnp.int32)
    gu, gn, gc, gs, ge = _sc_gather(u, n, c, s, e,
                                    user_table, news_table, cat_table,
                                    subcat_table, entity_table)
    out = _tc_fuse(gu, gn, gc, gs, ge, W, b.reshape(1, D))
    return out.reshape(B)
